# baseline (speedup 1.0000x reference)
import functools

import jax
import jax.numpy as jnp
from jax import lax
from jax.experimental import pallas as pl
from jax.experimental.pallas import tpu as pltpu
from jax.experimental.pallas import tpu_sc as plsc

N = 10000
E = 320000
H = 128
NC = 2
NS = 16
NW = NC * NS
CH = 128
EPT = 10240
NCHUNK = EPT // CH
EP = EPT * NW
NP = 10240
RPT = NP // NS

NBUF = 2
G = 16
NCH0 = 48
NCH1 = 112
NCHMAX = max(NCH0, NCH1)
EPX = (NCH0 + NCH1) * CH * NS


def _sc_agg_body(h_hbm, src_hbm, dst_hbm, zero_hbm, out_hbm,
                 src_v, dst_v, rows_v, acc_sh, sem0, sem1):
    cid = lax.axis_index("c")
    sid = lax.axis_index("s")
    t = cid * NS + sid
    pltpu.sync_copy(zero_hbm.at[pl.ds(sid * RPT, RPT)],
                    acc_sh.at[pl.ds(sid * RPT, RPT)])
    plsc.subcore_barrier()

    sems = (sem0, sem1)
    nph = jnp.where(cid == 0, NCH0 // G, NCH1 // G)

    def phase(p, carry):
        pltpu.sync_copy(src_hbm.at[t, pl.ds(p * G, G)], src_v)
        pltpu.sync_copy(dst_hbm.at[t, pl.ds(p * G, G)], dst_v)
        for b in range(NBUF):
            pltpu.async_copy(h_hbm.at[src_v.at[b]], rows_v.at[b], sems[b])

        def outer(i, c2):
            j0 = i * NBUF
            for b in range(NBUF):
                j = j0 + b
                pltpu.make_async_copy(h_hbm.at[src_v.at[j]],
                                      rows_v.at[b], sems[b]).wait()
                pltpu.sync_copy(rows_v.at[b], acc_sh.at[dst_v.at[j]],
                                add=True)

                @pl.when(j + NBUF < G)
                def _():
                    pltpu.async_copy(h_hbm.at[src_v.at[j + NBUF]],
                                     rows_v.at[b], sems[b])
            return c2

        lax.fori_loop(0, G // NBUF, outer, 0)
        return carry

    lax.fori_loop(0, nph, phase, 0)
    plsc.subcore_barrier()
    pltpu.sync_copy(acc_sh.at[pl.ds(sid * RPT, RPT)],
                    out_hbm.at[cid, pl.ds(sid * RPT, RPT)])


def _sc_edge_agg_body(ef_hbm, dst_hbm, zero_hbm, out_hbm, dst_v, ef_v, acc_sh):
    cid = lax.axis_index("c")
    sid = lax.axis_index("s")
    t = cid * NS + sid
    pltpu.sync_copy(dst_hbm.at[t], dst_v)
    pltpu.sync_copy(zero_hbm.at[pl.ds(sid * RPT, RPT)],
                    acc_sh.at[pl.ds(sid * RPT, RPT)])
    plsc.subcore_barrier()

    def body(j, carry):
        pltpu.sync_copy(ef_hbm.at[t, j], ef_v)
        pltpu.sync_copy(ef_v, acc_sh.at[dst_v.at[j]], add=True)
        return carry

    lax.fori_loop(0, NCHUNK, body, 0)
    plsc.subcore_barrier()
    pltpu.sync_copy(acc_sh.at[pl.ds(sid * RPT, RPT)],
                    out_hbm.at[cid, pl.ds(sid * RPT, RPT)])


@functools.lru_cache(maxsize=None)
def _sc_kernels():
    mesh = plsc.VectorSubcoreMesh(core_axis_name="c", subcore_axis_name="s")
    agg = pl.kernel(
        _sc_agg_body,
        mesh=mesh,
        out_type=jax.ShapeDtypeStruct((NC, NP, H), jnp.float32),
        scratch_types=[
            pltpu.VMEM((G, CH), jnp.int32),
            pltpu.VMEM((G, CH), jnp.int32),
            pltpu.VMEM((NBUF, CH, H), jnp.float32),
            pltpu.VMEM_SHARED((NP, H), jnp.float32),
            pltpu.SemaphoreType.DMA,
            pltpu.SemaphoreType.DMA,
        ],
    )
    edge_agg = pl.kernel(
        _sc_edge_agg_body,
        mesh=mesh,
        out_type=jax.ShapeDtypeStruct((NC, NP, 16), jnp.float32),
        scratch_types=[
            pltpu.VMEM((NCHUNK, CH), jnp.int32),
            pltpu.VMEM((CH, 16), jnp.float32),
            pltpu.VMEM_SHARED((NP, 16), jnp.float32),
        ],
    )
    return agg, edge_agg


BLK = 1024


def _lin_body(h_ref, p0_ref, p1_ref, he0_ref, he1_ref,
              ws_ref, wa_ref, we_ref, b_ref, o_ref):
    dn = (((1,), (1,)), ((), ()))
    hn = p0_ref[...] + p1_ref[...]
    he = he0_ref[...] + he1_ref[...]
    acc = lax.dot_general(h_ref[...], ws_ref[...], dn,
                          preferred_element_type=jnp.float32, precision=lax.Precision.HIGHEST)
    acc = acc + lax.dot_general(hn, wa_ref[...], dn,
                                preferred_element_type=jnp.float32, precision=lax.Precision.HIGHEST)
    acc = acc + he[:, 0:1] * we_ref[...]
    acc = acc + b_ref[...]
    o_ref[...] = jnp.maximum(acc, 0.0)


def _tc_linear(h, p0, p1, he0, he1, ws, wa, we, b):
    grid = (NP // BLK,)
    row = lambda i: (i, 0)
    rep = lambda i: (0, 0)
    return pl.pallas_call(
        _lin_body,
        grid=grid,
        in_specs=[
            pl.BlockSpec((BLK, H), row),
            pl.BlockSpec((BLK, H), row),
            pl.BlockSpec((BLK, H), row),
            pl.BlockSpec((BLK, 16), row),
            pl.BlockSpec((BLK, 16), row),
            pl.BlockSpec((H, H), rep),
            pl.BlockSpec((H, H), rep),
            pl.BlockSpec((1, H), rep),
            pl.BlockSpec((1, H), rep),
        ],
        out_specs=pl.BlockSpec((BLK, H), row),
        out_shape=jax.ShapeDtypeStruct((NP, H), jnp.float32),
    )(h, p0, p1, he0, he1, ws, wa, we, b)


def _tail_body(h_ref, p0_ref, p1_ref, he0_ref, he1_ref,
               ws_ref, wa_ref, we_ref, b_ref,
               w2_ref, b2_ref, w3_ref, b3_ref, o_ref):
    dn = (((1,), (1,)), ((), ()))
    hn = p0_ref[...] + p1_ref[...]
    he = he0_ref[...] + he1_ref[...]
    acc = lax.dot_general(h_ref[...], ws_ref[...], dn,
                          preferred_element_type=jnp.float32, precision=lax.Precision.HIGHEST)
    acc = acc + lax.dot_general(hn, wa_ref[...], dn,
                                preferred_element_type=jnp.float32, precision=lax.Precision.HIGHEST)
    acc = acc + he[:, 0:1] * we_ref[...] + b_ref[...]
    x = jnp.maximum(acc, 0.0)
    y = lax.dot_general(x, w2_ref[...], dn,
                        preferred_element_type=jnp.float32, precision=lax.Precision.HIGHEST) + b2_ref[...]
    y = jnp.maximum(y, 0.0)
    o_ref[...] = lax.dot_general(y, w3_ref[...], dn,
                                 preferred_element_type=jnp.float32, precision=lax.Precision.HIGHEST) + b3_ref[...]


def _pad_rows(a, rows):
    return jnp.pad(a, ((0, rows - a.shape[0]), (0, 0)))


def _tc_tail(h, p0, p1, he0, he1, ws, wa, we, b, w2, b2, w3, b3):
    grid = (NP // BLK,)
    row = lambda i: (i, 0)
    rep = lambda i: (0, 0)
    return pl.pallas_call(
        _tail_body,
        grid=grid,
        in_specs=[
            pl.BlockSpec((BLK, H), row),
            pl.BlockSpec((BLK, H), row),
            pl.BlockSpec((BLK, H), row),
            pl.BlockSpec((BLK, 16), row),
            pl.BlockSpec((BLK, 16), row),
            pl.BlockSpec((H, H), rep),
            pl.BlockSpec((H, H), rep),
            pl.BlockSpec((1, H), rep),
            pl.BlockSpec((1, H), rep),
            pl.BlockSpec((H, H), rep),
            pl.BlockSpec((1, H), rep),
            pl.BlockSpec((H, H), rep),
            pl.BlockSpec((1, H), rep),
        ],
        out_specs=pl.BlockSpec((BLK, H), row),
        out_shape=jax.ShapeDtypeStruct((NP, H), jnp.float32),
    )(h, p0, p1, he0, he1, ws, wa, we, b, w2, b2, w3, b3)


def kernel(node_feat, edge_feat, edge_index, params):
    src = edge_index[0]
    dst = edge_index[1]

    pad = EP - E
    dst_bal = jnp.concatenate(
        [dst, jnp.full((pad,), N, jnp.int32)]).reshape(NW, NCHUNK, CH)
    ef16 = jnp.concatenate(
        [jnp.broadcast_to(edge_feat, (E, 16)),
         jnp.zeros((pad, 16), jnp.float32)]).reshape(NW, NCHUNK, CH, 16)

    def asym(a, fill):
        padx = EPX - E
        ap = jnp.concatenate([a, jnp.full((padx,), fill, jnp.int32)])
        c0 = NS * NCH0 * CH
        p0 = ap[:c0].reshape(NS, NCH0, CH)
        p1 = ap[c0:].reshape(NS, NCH1, CH)
        p0 = jnp.pad(p0, ((0, 0), (0, NCHMAX - NCH0), (0, 0)),
                     constant_values=fill)
        p1 = jnp.pad(p1, ((0, 0), (0, NCHMAX - NCH1), (0, 0)),
                     constant_values=fill)
        return jnp.concatenate([p0, p1])

    src_p = asym(src, 0)
    dst_p = asym(dst, N)
    zeros128 = jnp.zeros((NP, H), jnp.float32)
    zeros16 = jnp.zeros((NP, 16), jnp.float32)

    sc_agg, sc_edge_agg = _sc_kernels()
    he_parts = sc_edge_agg(ef16, dst_bal, zeros16)
    he0, he1 = he_parts[0], he_parts[1]

    h = jnp.pad(node_feat, ((0, NP - N), (0, 0)))
    for name in ("conv1", "mid1", "mid2", "mid3", "mid4"):
        W, b = params[name]
        parts = sc_agg(h, src_p, dst_p, zeros128)
        h = _tc_linear(h, parts[0], parts[1], he0, he1,
                       W[:, :H], W[:, H:2 * H],
                       W[:, 2 * H].reshape(1, H), b.reshape(1, H))

    W1, b1 = params["reg1"]
    W2, b2 = params["reg2"]
    W3, b3 = params["reg3"]
    w3p = _pad_rows(W3, H)
    b3p = jnp.pad(b3, (0, H - 1)).reshape(1, H)
    parts = sc_agg(h, src_p, dst_p, zeros128)
    out = _tc_tail(h, parts[0], parts[1], he0, he1,
                   W1[:, :H], W1[:, H:2 * H],
                   W1[:, 2 * H].reshape(1, H), b1.reshape(1, H),
                   W2, b2.reshape(1, H), w3p, b3p)
    return out[:N, :1]


# speedup vs baseline: 1.2176x; 1.2176x over previous
import functools

import jax
import jax.numpy as jnp
from jax import lax
from jax.experimental import pallas as pl
from jax.experimental.pallas import tpu as pltpu
from jax.experimental.pallas import tpu_sc as plsc

N = 10000
E = 320000
H = 128
NC = 2
NS = 16
NW = NC * NS
CH = 128
EPT = 10240
NCHUNK = EPT // CH
EP = EPT * NW
NP = 10240
RPT = NP // NS

NBUF = 2
G = 16
NCH0 = 128
NCH1 = 32
NCHMAX = max(NCH0, NCH1)
EPX = (NCH0 + NCH1) * CH * NS


def _sc_agg_body(h_hbm, src_hbm, dst_hbm, zero_hbm, out_hbm,
                 src_v, dst_v, rows_v, acc_sh, sem0, sem1):
    cid = lax.axis_index("c")
    sid = lax.axis_index("s")
    t = cid * NS + sid
    pltpu.sync_copy(zero_hbm.at[pl.ds(sid * RPT, RPT)],
                    acc_sh.at[pl.ds(sid * RPT, RPT)])
    plsc.subcore_barrier()

    sems = (sem0, sem1)
    nph = jnp.where(cid == 0, NCH0 // G, NCH1 // G)

    def phase(p, carry):
        pltpu.sync_copy(src_hbm.at[t, pl.ds(p * G, G)], src_v)
        pltpu.sync_copy(dst_hbm.at[t, pl.ds(p * G, G)], dst_v)
        for b in range(NBUF):
            pltpu.async_copy(h_hbm.at[src_v.at[b]], rows_v.at[b], sems[b])

        def outer(i, c2):
            j0 = i * NBUF
            for b in range(NBUF):
                j = j0 + b
                pltpu.make_async_copy(h_hbm.at[src_v.at[j]],
                                      rows_v.at[b], sems[b]).wait()
                pltpu.sync_copy(rows_v.at[b], acc_sh.at[dst_v.at[j]],
                                add=True)

                @pl.when(j + NBUF < G)
                def _():
                    pltpu.async_copy(h_hbm.at[src_v.at[j + NBUF]],
                                     rows_v.at[b], sems[b])
            return c2

        lax.fori_loop(0, G // NBUF, outer, 0)
        return carry

    lax.fori_loop(0, nph, phase, 0)
    plsc.subcore_barrier()
    pltpu.sync_copy(acc_sh.at[pl.ds(sid * RPT, RPT)],
                    out_hbm.at[cid, pl.ds(sid * RPT, RPT)])


def _sc_edge_agg_body(ef_hbm, dst_hbm, zero_hbm, out_hbm, dst_v, ef_v, acc_sh):
    cid = lax.axis_index("c")
    sid = lax.axis_index("s")
    t = cid * NS + sid
    pltpu.sync_copy(dst_hbm.at[t], dst_v)
    pltpu.sync_copy(zero_hbm.at[pl.ds(sid * RPT, RPT)],
                    acc_sh.at[pl.ds(sid * RPT, RPT)])
    plsc.subcore_barrier()

    def body(j, carry):
        pltpu.sync_copy(ef_hbm.at[t, j], ef_v)
        pltpu.sync_copy(ef_v, acc_sh.at[dst_v.at[j]], add=True)
        return carry

    lax.fori_loop(0, NCHUNK, body, 0)
    plsc.subcore_barrier()
    pltpu.sync_copy(acc_sh.at[pl.ds(sid * RPT, RPT)],
                    out_hbm.at[cid, pl.ds(sid * RPT, RPT)])


@functools.lru_cache(maxsize=None)
def _sc_kernels():
    mesh = plsc.VectorSubcoreMesh(core_axis_name="c", subcore_axis_name="s")
    agg = pl.kernel(
        _sc_agg_body,
        mesh=mesh,
        out_type=jax.ShapeDtypeStruct((NC, NP, H), jnp.float32),
        scratch_types=[
            pltpu.VMEM((G, CH), jnp.int32),
            pltpu.VMEM((G, CH), jnp.int32),
            pltpu.VMEM((NBUF, CH, H), jnp.float32),
            pltpu.VMEM_SHARED((NP, H), jnp.float32),
            pltpu.SemaphoreType.DMA,
            pltpu.SemaphoreType.DMA,
        ],
    )
    edge_agg = pl.kernel(
        _sc_edge_agg_body,
        mesh=mesh,
        out_type=jax.ShapeDtypeStruct((NC, NP, 16), jnp.float32),
        scratch_types=[
            pltpu.VMEM((NCHUNK, CH), jnp.int32),
            pltpu.VMEM((CH, 16), jnp.float32),
            pltpu.VMEM_SHARED((NP, 16), jnp.float32),
        ],
    )
    return agg, edge_agg


BLK = 1024


def _lin_body(h_ref, p0_ref, p1_ref, he0_ref, he1_ref,
              ws_ref, wa_ref, we_ref, b_ref, o_ref):
    dn = (((1,), (1,)), ((), ()))
    hn = p0_ref[...] + p1_ref[...]
    he = he0_ref[...] + he1_ref[...]
    acc = lax.dot_general(h_ref[...], ws_ref[...], dn,
                          preferred_element_type=jnp.float32, precision=lax.Precision.HIGHEST)
    acc = acc + lax.dot_general(hn, wa_ref[...], dn,
                                preferred_element_type=jnp.float32, precision=lax.Precision.HIGHEST)
    acc = acc + he[:, 0:1] * we_ref[...]
    acc = acc + b_ref[...]
    o_ref[...] = jnp.maximum(acc, 0.0)


def _tc_linear(h, p0, p1, he0, he1, ws, wa, we, b):
    grid = (NP // BLK,)
    row = lambda i: (i, 0)
    rep = lambda i: (0, 0)
    return pl.pallas_call(
        _lin_body,
        grid=grid,
        in_specs=[
            pl.BlockSpec((BLK, H), row),
            pl.BlockSpec((BLK, H), row),
            pl.BlockSpec((BLK, H), row),
            pl.BlockSpec((BLK, 16), row),
            pl.BlockSpec((BLK, 16), row),
            pl.BlockSpec((H, H), rep),
            pl.BlockSpec((H, H), rep),
            pl.BlockSpec((1, H), rep),
            pl.BlockSpec((1, H), rep),
        ],
        out_specs=pl.BlockSpec((BLK, H), row),
        out_shape=jax.ShapeDtypeStruct((NP, H), jnp.float32),
    )(h, p0, p1, he0, he1, ws, wa, we, b)


def _tail_body(h_ref, p0_ref, p1_ref, he0_ref, he1_ref,
               ws_ref, wa_ref, we_ref, b_ref,
               w2_ref, b2_ref, w3_ref, b3_ref, o_ref):
    dn = (((1,), (1,)), ((), ()))
    hn = p0_ref[...] + p1_ref[...]
    he = he0_ref[...] + he1_ref[...]
    acc = lax.dot_general(h_ref[...], ws_ref[...], dn,
                          preferred_element_type=jnp.float32, precision=lax.Precision.HIGHEST)
    acc = acc + lax.dot_general(hn, wa_ref[...], dn,
                                preferred_element_type=jnp.float32, precision=lax.Precision.HIGHEST)
    acc = acc + he[:, 0:1] * we_ref[...] + b_ref[...]
    x = jnp.maximum(acc, 0.0)
    y = lax.dot_general(x, w2_ref[...], dn,
                        preferred_element_type=jnp.float32, precision=lax.Precision.HIGHEST) + b2_ref[...]
    y = jnp.maximum(y, 0.0)
    o_ref[...] = lax.dot_general(y, w3_ref[...], dn,
                                 preferred_element_type=jnp.float32, precision=lax.Precision.HIGHEST) + b3_ref[...]


def _pad_rows(a, rows):
    return jnp.pad(a, ((0, rows - a.shape[0]), (0, 0)))


def _tc_tail(h, p0, p1, he0, he1, ws, wa, we, b, w2, b2, w3, b3):
    grid = (NP // BLK,)
    row = lambda i: (i, 0)
    rep = lambda i: (0, 0)
    return pl.pallas_call(
        _tail_body,
        grid=grid,
        in_specs=[
            pl.BlockSpec((BLK, H), row),
            pl.BlockSpec((BLK, H), row),
            pl.BlockSpec((BLK, H), row),
            pl.BlockSpec((BLK, 16), row),
            pl.BlockSpec((BLK, 16), row),
            pl.BlockSpec((H, H), rep),
            pl.BlockSpec((H, H), rep),
            pl.BlockSpec((1, H), rep),
            pl.BlockSpec((1, H), rep),
            pl.BlockSpec((H, H), rep),
            pl.BlockSpec((1, H), rep),
            pl.BlockSpec((H, H), rep),
            pl.BlockSpec((1, H), rep),
        ],
        out_specs=pl.BlockSpec((BLK, H), row),
        out_shape=jax.ShapeDtypeStruct((NP, H), jnp.float32),
    )(h, p0, p1, he0, he1, ws, wa, we, b, w2, b2, w3, b3)


def kernel(node_feat, edge_feat, edge_index, params):
    src = edge_index[0]
    dst = edge_index[1]

    pad = EP - E
    dst_bal = jnp.concatenate(
        [dst, jnp.full((pad,), N, jnp.int32)]).reshape(NW, NCHUNK, CH)
    ef16 = jnp.concatenate(
        [jnp.broadcast_to(edge_feat, (E, 16)),
         jnp.zeros((pad, 16), jnp.float32)]).reshape(NW, NCHUNK, CH, 16)

    def asym(a, fill):
        padx = EPX - E
        ap = jnp.concatenate([a, jnp.full((padx,), fill, jnp.int32)])
        c0 = NS * NCH0 * CH
        p0 = ap[:c0].reshape(NS, NCH0, CH)
        p1 = ap[c0:].reshape(NS, NCH1, CH)
        p0 = jnp.pad(p0, ((0, 0), (0, NCHMAX - NCH0), (0, 0)),
                     constant_values=fill)
        p1 = jnp.pad(p1, ((0, 0), (0, NCHMAX - NCH1), (0, 0)),
                     constant_values=fill)
        return jnp.concatenate([p0, p1])

    src_p = asym(src, 0)
    dst_p = asym(dst, N)
    zeros128 = jnp.zeros((NP, H), jnp.float32)
    zeros16 = jnp.zeros((NP, 16), jnp.float32)

    sc_agg, sc_edge_agg = _sc_kernels()
    he_parts = sc_edge_agg(ef16, dst_bal, zeros16)
    he0, he1 = he_parts[0], he_parts[1]

    h = jnp.pad(node_feat, ((0, NP - N), (0, 0)))
    for name in ("conv1", "mid1", "mid2", "mid3", "mid4"):
        W, b = params[name]
        parts = sc_agg(h, src_p, dst_p, zeros128)
        h = _tc_linear(h, parts[0], parts[1], he0, he1,
                       W[:, :H], W[:, H:2 * H],
                       W[:, 2 * H].reshape(1, H), b.reshape(1, H))

    W1, b1 = params["reg1"]
    W2, b2 = params["reg2"]
    W3, b3 = params["reg3"]
    w3p = _pad_rows(W3, H)
    b3p = jnp.pad(b3, (0, H - 1)).reshape(1, H)
    parts = sc_agg(h, src_p, dst_p, zeros128)
    out = _tc_tail(h, parts[0], parts[1], he0, he1,
                   W1[:, :H], W1[:, H:2 * H],
                   W1[:, 2 * H].reshape(1, H), b1.reshape(1, H),
                   W2, b2.reshape(1, H), w3p, b3p)
    return out[:N, :1]
